# baseline (speedup 1.0000x reference)
import jax
import jax.numpy as jnp
from jax.experimental import pallas as pl
from jax.experimental.pallas import tpu as pltpu

_N = 10000
_F = 128
_BM = 80
_NBUF = 5
_STEPS = _N // _BM
_ROUNDS = _STEPS // _NBUF
_RROWS = _NBUF * _BM


def _body(adj_hbm, x_ref, w_ref, out_hbm, buf_ref, sem, sup_ref,
          ostg_ref, osem):
    def _start(step, slot):
        pltpu.make_async_copy(
            adj_hbm.at[pl.ds(step * _BM, _BM), :],
            buf_ref.at[slot],
            sem.at[slot],
        ).start()

    for slot in range(_NBUF):
        _start(slot, slot)

    sup_ref[...] = jax.lax.dot_general(
        x_ref[...], w_ref[...], (((1,), (0,)), ((), ())),
        preferred_element_type=jnp.float32,
        precision=jax.lax.Precision.HIGHEST).astype(jnp.bfloat16)

    def _out_copy(b, pb):
        return pltpu.make_async_copy(
            ostg_ref.at[pl.ds(pb * _RROWS, _RROWS), :],
            out_hbm.at[pl.ds(b * _RROWS, _RROWS), :],
            osem.at[pb],
        )

    def _round(b, carry):
        pb = jax.lax.rem(b, 2)

        @pl.when(b >= 2)
        def _():
            _out_copy(b, pb).wait()

        for slot in range(_NBUF):
            i = b * _NBUF + slot
            pltpu.make_async_copy(
                adj_hbm.at[pl.ds(i * _BM, _BM), :],
                buf_ref.at[slot],
                sem.at[slot],
            ).wait()
            acc = jax.lax.dot_general(
                buf_ref[slot], sup_ref[...], (((1,), (0,)), ((), ())),
                preferred_element_type=jnp.float32)
            ostg_ref[pl.ds(pb * _RROWS + slot * _BM, _BM), :] = (
                jnp.maximum(acc, 0.0))

            @pl.when(i + _NBUF < _STEPS)
            def _():
                _start(i + _NBUF, slot)

        _out_copy(b, pb).start()
        return carry

    jax.lax.fori_loop(0, _ROUNDS, _round, 0)

    _out_copy(_ROUNDS - 2, jax.lax.rem(_ROUNDS - 2, 2)).wait()
    _out_copy(_ROUNDS - 1, jax.lax.rem(_ROUNDS - 1, 2)).wait()


def kernel(adj, x_input, weight):
    return pl.pallas_call(
        _body,
        in_specs=[pl.BlockSpec(memory_space=pl.ANY),
                  pl.BlockSpec((_N, _F), lambda: (0, 0)),
                  pl.BlockSpec((_F, _F), lambda: (0, 0))],
        out_specs=pl.BlockSpec(memory_space=pl.ANY),
        out_shape=jax.ShapeDtypeStruct((_N, _F), jnp.float32),
        scratch_shapes=[
            pltpu.VMEM((_NBUF, _BM, _N), jnp.float32),
            pltpu.SemaphoreType.DMA((_NBUF,)),
            pltpu.VMEM((_N, _F), jnp.bfloat16),
            pltpu.VMEM((2 * _RROWS, _F), jnp.float32),
            pltpu.SemaphoreType.DMA((2,)),
        ],
        compiler_params=pltpu.CompilerParams(
            dimension_semantics=()),
    )(adj, x_input, weight)


# speedup vs baseline: 1.0014x; 1.0014x over previous
import jax
import jax.numpy as jnp
from jax.experimental import pallas as pl
from jax.experimental.pallas import tpu as pltpu

_N = 10000
_F = 128
_BM = 80
_NBUF = 5
_STEPS = _N // _BM


def _body(adj_hbm, x_ref, w_ref, out_ref, buf_ref, sem, sup_ref):
    def _start(step, slot):
        pltpu.make_async_copy(
            adj_hbm.at[pl.ds(step * _BM, _BM), :],
            buf_ref.at[slot],
            sem.at[slot],
        ).start()

    for slot in range(_NBUF):
        _start(slot, slot)

    sup_ref[...] = jax.lax.dot_general(
        x_ref[...], w_ref[...], (((1,), (0,)), ((), ())),
        preferred_element_type=jnp.float32,
        precision=jax.lax.Precision.HIGHEST).astype(jnp.bfloat16)

    def _round(b, carry):
        for slot in range(_NBUF):
            i = b * _NBUF + slot
            pltpu.make_async_copy(
                adj_hbm.at[pl.ds(i * _BM, _BM), :],
                buf_ref.at[slot],
                sem.at[slot],
            ).wait()
            acc = jax.lax.dot_general(
                buf_ref[slot], sup_ref[...], (((1,), (0,)), ((), ())),
                preferred_element_type=jnp.float32)
            out_ref[pl.ds(i * _BM, _BM), :] = jnp.maximum(acc, 0.0)

            @pl.when(i + _NBUF < _STEPS)
            def _():
                _start(i + _NBUF, slot)

        return carry

    jax.lax.fori_loop(0, _STEPS // _NBUF, _round, 0)


def kernel(adj, x_input, weight):
    return pl.pallas_call(
        _body,
        in_specs=[pl.BlockSpec(memory_space=pl.ANY),
                  pl.BlockSpec((_N, _F), lambda: (0, 0)),
                  pl.BlockSpec((_F, _F), lambda: (0, 0))],
        out_specs=pl.BlockSpec((_N, _F), lambda: (0, 0)),
        out_shape=jax.ShapeDtypeStruct((_N, _F), jnp.float32),
        scratch_shapes=[
            pltpu.VMEM((_NBUF, _BM, _N), jnp.float32),
            pltpu.SemaphoreType.DMA((_NBUF,)),
            pltpu.VMEM((_N, _F), jnp.bfloat16),
        ],
        compiler_params=pltpu.CompilerParams(
            dimension_semantics=()),
    )(adj, x_input, weight)
